# baseline (speedup 1.0000x reference)
import functools

import jax
import jax.numpy as jnp
from jax import lax
from jax.experimental import pallas as pl
from jax.experimental.pallas import tpu as pltpu
from jax.experimental.pallas import tpu_sc as plsc

_B = 16384
_NF = 26
_V = 100000
_EMB = 16
_NC = _NF * _EMB
_NW = 32
_JROWS = _B // 128
_BM = 512
_GROUPS = ((0, 14), (14, 26))

_MESH = dict(core_axis_name="c", subcore_axis_name="s")


def _fire_col(src_ref, base, idx_row_fn, col_v, buf, sem):
    def grp(s, carry):
        for kk in range(16):
            j = s * 16 + kk
            pltpu.async_copy(
                src_ref.at[pl.ds(base, _V)].at[idx_row_fn(j)],
                col_v.at[buf, j], sem)
        return carry

    lax.fori_loop(0, _JROWS // 16, grp, 0)


def _drain_col(src_ref, col_v, buf, sem):
    pltpu.make_async_copy(src_ref.at[pl.ds(0, _B)], col_v.at[buf], sem).wait()


def _sc_gather_emb(sparse_t3, tab_g, f_start, nf):
    ncols = nf * _EMB
    cpw = ncols // _NW

    @functools.partial(
        pl.kernel,
        mesh=plsc.VectorSubcoreMesh(**_MESH),
        compiler_params=pltpu.CompilerParams(use_tc_tiling_on_sc=False),
        out_type=jax.ShapeDtypeStruct((ncols // 8, 128, 1024), jnp.float32),
        scratch_types=[
            pltpu.VMEM((2, _JROWS, 128), jnp.int32),
            pltpu.VMEM((2, 128, 128), jnp.float32),
            pltpu.SemaphoreType.DMA,
            pltpu.SemaphoreType.DMA,
        ],
    )
    def k(sp_hbm, tab_hbm, embt_out, idx_v, col_v, s0, s1):
        wid = lax.axis_index("s") * 2 + lax.axis_index("c")
        c0 = wid * cpw
        fl0 = lax.shift_right_logical(c0, 4)
        fl_last = lax.shift_right_logical(c0 + (cpw - 1), 4)
        pltpu.sync_copy(sp_hbm.at[f_start + fl0], idx_v.at[0])
        pltpu.sync_copy(sp_hbm.at[f_start + fl_last], idx_v.at[1])
        sems = [s0, s1]

        def write_col(buf, c):
            pr = lax.shift_right_logical(c, 3)
            psub = lax.bitwise_and(c, 7)
            pltpu.sync_copy(col_v.at[buf],
                            embt_out.at[pr, pl.ds(0, 128),
                                        pl.ds(psub * 128, 128)])

        for t in range(cpw):
            c = c0 + t
            sel = lax.shift_right_logical(c, 4) - fl0
            _fire_col(tab_hbm, c * _V, lambda j: idx_v.at[sel, j],
                      col_v, t % 2, sems[t % 2])
            if t:
                _drain_col(tab_hbm, col_v, (t - 1) % 2, sems[(t - 1) % 2])
                write_col((t - 1) % 2, c - 1)
        _drain_col(tab_hbm, col_v, (cpw - 1) % 2, sems[(cpw - 1) % 2])
        write_col((cpw - 1) % 2, c0 + cpw - 1)

    return k(sparse_t3, tab_g)


def _sc_gather_w(sparse_t3, wtab_f):
    @functools.partial(
        pl.kernel,
        mesh=plsc.VectorSubcoreMesh(**_MESH),
        compiler_params=pltpu.CompilerParams(use_tc_tiling_on_sc=False),
        out_type=jax.ShapeDtypeStruct((_NF, 128, 128), jnp.float32),
        scratch_types=[
            pltpu.VMEM((_JROWS, 128), jnp.int32),
            pltpu.VMEM((1, 128, 128), jnp.float32),
            pltpu.SemaphoreType.DMA,
        ],
    )
    def k(sp_hbm, wtab_hbm, wt_out, idx_v, col_v, sem):
        wid = lax.axis_index("s") * 2 + lax.axis_index("c")

        @pl.when(wid < _NF)
        def _():
            pltpu.sync_copy(sp_hbm.at[wid], idx_v)
            _fire_col(wtab_hbm, wid * _V, lambda j: idx_v.at[j],
                      col_v, 0, sem)
            _drain_col(wtab_hbm, col_v, 0, sem)
            pltpu.sync_copy(col_v.at[0], wt_out.at[wid])

    return k(sparse_t3, wtab_f)


def _dot0(a, b):
    return lax.dot_general(a, b, (((0,), (0,)), ((), ())),
                           preferred_element_type=jnp.float32)


def _tc_head(embts, da, wt, W1as, W1c, b1s, W2s, b2s, W3, fmW, bias3, Ms,
             oness, interpret=False):
    grid = (_B // _BM,)
    ng = len(embts)

    def full(a):
        return pl.BlockSpec(a.shape, lambda i: tuple(0 for _ in a.shape))

    ones26 = jnp.ones((_NF, 1), jnp.float32)

    def body(*refs):
        embt_refs = refs[:ng]
        (da_ref, wt_ref) = refs[ng:ng + 2]
        W1a_refs = refs[ng + 2:2 * ng + 2]
        (W1c_ref, b1_ref, W2_ref, b2_ref, W3_ref, fmW_ref,
         b3_ref) = refs[2 * ng + 2:2 * ng + 9]
        M_refs = refs[2 * ng + 9:3 * ng + 9]
        ones_refs = refs[3 * ng + 9:4 * ng + 9]
        o26_ref = refs[4 * ng + 9]
        out_ref = refs[4 * ng + 10]

        da_blk = da_ref[...]
        h = (jnp.dot(da_blk, W1c_ref[...],
                     preferred_element_type=jnp.float32) + b1_ref[...])
        s = None
        q = None
        for g in range(ng):
            eg = embt_refs[g][...]
            h = h + _dot0(eg, W1a_refs[g][...])
            sg = _dot0(eg, M_refs[g][...])
            qg = _dot0(eg * eg, ones_refs[g][...])
            s = sg if s is None else s + sg
            q = qg if q is None else q + qg
        h = jnp.maximum(h, 0.0)
        h = (jnp.dot(h, W2_ref[...], preferred_element_type=jnp.float32)
             + b2_ref[...])
        h = jnp.maximum(h, 0.0)
        deep = jnp.dot(h, W3_ref[...], preferred_element_type=jnp.float32)
        fm1 = (_dot0(wt_ref[...], o26_ref[...])
               + jnp.dot(da_blk, fmW_ref[...],
                         preferred_element_type=jnp.float32))
        fm2 = 0.5 * (jnp.sum(s * s, axis=1, keepdims=True) - q)
        out_ref[...] = jax.nn.sigmoid(deep + fm1 + fm2 + b3_ref[...])

    in_specs = (
        [pl.BlockSpec((e.shape[0], _BM), lambda i: (0, i)) for e in embts]
        + [pl.BlockSpec((_BM, da.shape[1]), lambda i: (i, 0)),
           pl.BlockSpec((_NF, _BM), lambda i: (0, i))]
        + [full(w) for w in W1as]
        + [full(W1c), full(b1s), full(W2s), full(b2s), full(W3), full(fmW),
           full(bias3)]
        + [full(m) for m in Ms]
        + [full(o) for o in oness]
        + [full(ones26)]
    )
    return pl.pallas_call(
        body,
        grid=grid,
        in_specs=in_specs,
        out_specs=pl.BlockSpec((_BM, 1), lambda i: (i, 0)),
        out_shape=jax.ShapeDtypeStruct((_B, 1), jnp.float32),
        interpret=interpret,
    )(*embts, da, wt, *W1as, W1c, b1s, W2s, b2s, W3, fmW, bias3, *Ms, *oness,
      ones26)


def kernel(dense_inputs, sparse_inputs, bge_inputs, emb_tables, w_tables,
           fm_dense_W, fm_dense_b, W1, b1, g1, be1, W2, b2, g2, be2, W3, b3):
    sparse_t3 = sparse_inputs.astype(jnp.int32).T.reshape(_NF, _JROWS, 128)

    t3 = emb_tables.transpose(0, 2, 1)
    embts = []
    tab_prev = None
    x4_prev = None
    for (fa, fb) in _GROUPS:
        nf = fb - fa
        sl = t3[fa:fb]
        if tab_prev is not None:
            sl, _ = lax.optimization_barrier((sl, tab_prev))
        tab_g = sl.reshape(nf * _EMB * _V)
        sp = sparse_t3
        if x4_prev is not None:
            sp, _ = lax.optimization_barrier((sp, x4_prev))
        x4 = _sc_gather_emb(sp, tab_g, fa, nf)
        tab_prev, x4_prev = tab_g, x4
        ncols = nf * _EMB
        embt_g = x4.reshape(ncols // 8, 128, 8, 128).transpose(
            0, 2, 1, 3).reshape(ncols, _B)
        embts.append(embt_g)

    w_tables_b, _ = lax.optimization_barrier((w_tables, tab_prev))
    wtab_f = w_tables_b.reshape(_NF * _V)
    sparse_t3_b, _ = lax.optimization_barrier((sparse_t3, x4_prev))
    wt = _sc_gather_w(sparse_t3_b, wtab_f).reshape(_NF, _B)

    da = jnp.concatenate([dense_inputs, bge_inputs], axis=1)
    inv = 1.0 / jnp.sqrt(jnp.float32(1.0 + 1e-5))
    sc1 = g1 * inv
    W1s = W1 * sc1[None, :]
    b1s = (b1 * sc1 + be1)[None, :]
    sc2 = g2 * inv
    W2s = W2 * sc2[None, :]
    b2s = (b2 * sc2 + be2)[None, :]
    bias3 = (b3 + fm_dense_b).reshape(1, 1)
    M = jnp.tile(jnp.eye(_EMB, dtype=jnp.float32), (_NF, 1))

    W1as, Ms, oness = [], [], []
    for (fa, fb) in _GROUPS:
        W1as.append(W1s[fa * _EMB:fb * _EMB])
        Ms.append(M[fa * _EMB:fb * _EMB])
        oness.append(jnp.ones(((fb - fa) * _EMB, 1), jnp.float32))
    W1c = W1s[_NC:]

    return _tc_head(embts, da, wt, W1as, W1c, b1s, W2s, b2s, W3,
                    fmW=fm_dense_W, bias3=bias3, Ms=Ms, oness=oness)


# speedup vs baseline: 1.3869x; 1.3869x over previous
import functools

import jax
import jax.numpy as jnp
from jax import lax
from jax.experimental import pallas as pl
from jax.experimental.pallas import tpu as pltpu
from jax.experimental.pallas import tpu_sc as plsc

_B = 16384
_NF = 26
_V = 100000
_EMB = 16
_NC = _NF * _EMB
_NW = 32
_JROWS = _B // 128
_BM = 512
_GROUPS = ((0, 26),)

_MESH = dict(core_axis_name="c", subcore_axis_name="s")


def _fire_col(src_ref, base, idx_row_fn, col_v, buf, sem):
    def grp(s, carry):
        for kk in range(16):
            j = s * 16 + kk
            pltpu.async_copy(
                src_ref.at[pl.ds(base, _V)].at[idx_row_fn(j)],
                col_v.at[buf, j], sem)
        return carry

    lax.fori_loop(0, _JROWS // 16, grp, 0)


def _drain_col(src_ref, col_v, buf, sem):
    pltpu.make_async_copy(src_ref.at[pl.ds(0, _B)], col_v.at[buf], sem).wait()


def _sc_gather_emb(sparse_t3, tab_g, f_start, nf):
    ncols = nf * _EMB
    cpw = ncols // _NW

    @functools.partial(
        pl.kernel,
        mesh=plsc.VectorSubcoreMesh(**_MESH),
        compiler_params=pltpu.CompilerParams(use_tc_tiling_on_sc=False),
        out_type=jax.ShapeDtypeStruct((ncols // 8, 128, 1024), jnp.float32),
        scratch_types=[
            pltpu.VMEM((2, _JROWS, 128), jnp.int32),
            pltpu.VMEM((2, 128, 128), jnp.float32),
            pltpu.SemaphoreType.DMA,
            pltpu.SemaphoreType.DMA,
        ],
    )
    def k(sp_hbm, tab_hbm, embt_out, idx_v, col_v, s0, s1):
        wid = lax.axis_index("s") * 2 + lax.axis_index("c")
        c0 = wid * cpw
        fl0 = lax.shift_right_logical(c0, 4)
        fl_last = lax.shift_right_logical(c0 + (cpw - 1), 4)
        pltpu.sync_copy(sp_hbm.at[f_start + fl0], idx_v.at[0])
        pltpu.sync_copy(sp_hbm.at[f_start + fl_last], idx_v.at[1])
        sems = [s0, s1]

        def write_col(buf, c):
            pr = lax.shift_right_logical(c, 3)
            psub = lax.bitwise_and(c, 7)
            pltpu.sync_copy(col_v.at[buf],
                            embt_out.at[pr, pl.ds(0, 128),
                                        pl.ds(psub * 128, 128)])

        for t in range(cpw):
            c = c0 + t
            sel = lax.shift_right_logical(c, 4) - fl0
            _fire_col(tab_hbm, c * _V, lambda j: idx_v.at[sel, j],
                      col_v, t % 2, sems[t % 2])
            if t:
                _drain_col(tab_hbm, col_v, (t - 1) % 2, sems[(t - 1) % 2])
                write_col((t - 1) % 2, c - 1)
        _drain_col(tab_hbm, col_v, (cpw - 1) % 2, sems[(cpw - 1) % 2])
        write_col((cpw - 1) % 2, c0 + cpw - 1)

    return k(sparse_t3, tab_g)


def _sc_gather_w(sparse_t3, wtab_f):
    @functools.partial(
        pl.kernel,
        mesh=plsc.VectorSubcoreMesh(**_MESH),
        compiler_params=pltpu.CompilerParams(use_tc_tiling_on_sc=False),
        out_type=jax.ShapeDtypeStruct((_NF, 128, 128), jnp.float32),
        scratch_types=[
            pltpu.VMEM((_JROWS, 128), jnp.int32),
            pltpu.VMEM((1, 128, 128), jnp.float32),
            pltpu.SemaphoreType.DMA,
        ],
    )
    def k(sp_hbm, wtab_hbm, wt_out, idx_v, col_v, sem):
        wid = lax.axis_index("s") * 2 + lax.axis_index("c")

        @pl.when(wid < _NF)
        def _():
            pltpu.sync_copy(sp_hbm.at[wid], idx_v)
            _fire_col(wtab_hbm, wid * _V, lambda j: idx_v.at[j],
                      col_v, 0, sem)
            _drain_col(wtab_hbm, col_v, 0, sem)
            pltpu.sync_copy(col_v.at[0], wt_out.at[wid])

    return k(sparse_t3, wtab_f)


def _dot0(a, b):
    return lax.dot_general(a, b, (((0,), (0,)), ((), ())),
                           preferred_element_type=jnp.float32)


def _tc_head(embts, da, wt, W1as, W1c, b1s, W2s, b2s, W3, fmW, bias3, Ms,
             oness, interpret=False):
    grid = (_B // _BM,)
    ng = len(embts)

    def full(a):
        return pl.BlockSpec(a.shape, lambda i: tuple(0 for _ in a.shape))

    ones26 = jnp.ones((_NF, 1), jnp.float32)

    def body(*refs):
        embt_refs = refs[:ng]
        (da_ref, wt_ref) = refs[ng:ng + 2]
        W1a_refs = refs[ng + 2:2 * ng + 2]
        (W1c_ref, b1_ref, W2_ref, b2_ref, W3_ref, fmW_ref,
         b3_ref) = refs[2 * ng + 2:2 * ng + 9]
        M_refs = refs[2 * ng + 9:3 * ng + 9]
        ones_refs = refs[3 * ng + 9:4 * ng + 9]
        o26_ref = refs[4 * ng + 9]
        out_ref = refs[4 * ng + 10]

        da_blk = da_ref[...]
        h = (jnp.dot(da_blk, W1c_ref[...],
                     preferred_element_type=jnp.float32) + b1_ref[...])
        s = None
        q = None
        for g in range(ng):
            eg = embt_refs[g][...]
            h = h + _dot0(eg, W1a_refs[g][...])
            sg = _dot0(eg, M_refs[g][...])
            qg = _dot0(eg * eg, ones_refs[g][...])
            s = sg if s is None else s + sg
            q = qg if q is None else q + qg
        h = jnp.maximum(h, 0.0)
        h = (jnp.dot(h, W2_ref[...], preferred_element_type=jnp.float32)
             + b2_ref[...])
        h = jnp.maximum(h, 0.0)
        deep = jnp.dot(h, W3_ref[...], preferred_element_type=jnp.float32)
        fm1 = (_dot0(wt_ref[...], o26_ref[...])
               + jnp.dot(da_blk, fmW_ref[...],
                         preferred_element_type=jnp.float32))
        fm2 = 0.5 * (jnp.sum(s * s, axis=1, keepdims=True) - q)
        out_ref[...] = jax.nn.sigmoid(deep + fm1 + fm2 + b3_ref[...])

    in_specs = (
        [pl.BlockSpec((e.shape[0], _BM), lambda i: (0, i)) for e in embts]
        + [pl.BlockSpec((_BM, da.shape[1]), lambda i: (i, 0)),
           pl.BlockSpec((_NF, _BM), lambda i: (0, i))]
        + [full(w) for w in W1as]
        + [full(W1c), full(b1s), full(W2s), full(b2s), full(W3), full(fmW),
           full(bias3)]
        + [full(m) for m in Ms]
        + [full(o) for o in oness]
        + [full(ones26)]
    )
    return pl.pallas_call(
        body,
        grid=grid,
        in_specs=in_specs,
        out_specs=pl.BlockSpec((_BM, 1), lambda i: (i, 0)),
        out_shape=jax.ShapeDtypeStruct((_B, 1), jnp.float32),
        interpret=interpret,
    )(*embts, da, wt, *W1as, W1c, b1s, W2s, b2s, W3, fmW, bias3, *Ms, *oness,
      ones26)


def kernel(dense_inputs, sparse_inputs, bge_inputs, emb_tables, w_tables,
           fm_dense_W, fm_dense_b, W1, b1, g1, be1, W2, b2, g2, be2, W3, b3):
    sparse_t3 = sparse_inputs.astype(jnp.int32).T.reshape(_NF, _JROWS, 128)

    embts = []
    for (fa, fb) in _GROUPS:
        nf = fb - fa
        tab_g = emb_tables[fa:fb].transpose(0, 2, 1).reshape(nf * _EMB * _V)
        x4 = _sc_gather_emb(sparse_t3, tab_g, fa, nf)
        ncols = nf * _EMB
        embt_g = x4.reshape(ncols // 8, 128, 8, 128).transpose(
            0, 2, 1, 3).reshape(ncols, _B)
        embts.append(embt_g)

    w_tables_b, _ = lax.optimization_barrier((w_tables, tab_g))
    wtab_f = w_tables_b.reshape(_NF * _V)
    sparse_t3_b, _ = lax.optimization_barrier((sparse_t3, embts[0]))
    wt = _sc_gather_w(sparse_t3_b, wtab_f).reshape(_NF, _B)

    da = jnp.concatenate([dense_inputs, bge_inputs], axis=1)
    inv = 1.0 / jnp.sqrt(jnp.float32(1.0 + 1e-5))
    sc1 = g1 * inv
    W1s = W1 * sc1[None, :]
    b1s = (b1 * sc1 + be1)[None, :]
    sc2 = g2 * inv
    W2s = W2 * sc2[None, :]
    b2s = (b2 * sc2 + be2)[None, :]
    bias3 = (b3 + fm_dense_b).reshape(1, 1)
    M = jnp.tile(jnp.eye(_EMB, dtype=jnp.float32), (_NF, 1))

    W1as, Ms, oness = [], [], []
    for (fa, fb) in _GROUPS:
        W1as.append(W1s[fa * _EMB:fb * _EMB])
        Ms.append(M[fa * _EMB:fb * _EMB])
        oness.append(jnp.ones(((fb - fa) * _EMB, 1), jnp.float32))
    W1c = W1s[_NC:]

    return _tc_head(embts, da, wt, W1as, W1c, b1s, W2s, b2s, W3,
                    fmW=fm_dense_W, bias3=bias3, Ms=Ms, oness=oness)
